# baseline (speedup 1.0000x reference)
import functools
import math

import jax
import jax.numpy as jnp
from jax import lax
from jax.experimental import pallas as pl
from jax.experimental.pallas import tpu as pltpu
from jax.experimental.pallas import tpu_sc as plsc

B = 4
S = 4096
D = 1024
SCALE = math.sqrt(D)

NC = 2
NS = 16
NW = NC * NS
POS_PER_W = S // NW
CH = 4
NCHUNK = POS_PER_W // CH
G = D // 16

_mesh = plsc.VectorSubcoreMesh(
    core_axis_name="c", subcore_axis_name="s", num_cores=NC, num_subcores=NS
)


@functools.partial(
    pl.kernel,
    out_type=(
        jax.ShapeDtypeStruct((B, S, D), jnp.float32),
        jax.ShapeDtypeStruct((B, NW, D), jnp.float32),
    ),
    mesh=_mesh,
    scratch_types=[
        pltpu.VMEM((NCHUNK * B * CH,), jnp.int32),
        pltpu.VMEM((B * CH, D), jnp.float32),
        pltpu.VMEM((B * CH, D), jnp.float32),
        pltpu.VMEM((B * CH, D), jnp.float32),
        pltpu.VMEM((CH, D), jnp.float32),
        pltpu.VMEM((CH, D), jnp.float32),
        pltpu.VMEM((B * CH, D), jnp.float32),
        pltpu.VMEM((B * CH, D), jnp.float32),
        pltpu.VMEM((D,), jnp.float32),
        pltpu.VMEM((B, D), jnp.float32),
        pltpu.SemaphoreType.DMA,
        pltpu.SemaphoreType.DMA,
        pltpu.SemaphoreType.DMA,
        pltpu.SemaphoreType.DMA,
        pltpu.SemaphoreType.DMA,
        pltpu.SemaphoreType.DMA,
        pltpu.SemaphoreType.DMA,
    ],
)
def _sc_embed(ids_hbm, tok_hbm, pos_hbm, mod_hbm,
              out_hbm, part_hbm,
              idx_v, tokA, tokB, tokC, pos0, pos1, o0, o1, mod_v, acc_v,
              gtA, gtB, gtC, gp0, gp1, go0, go1):
    wid = lax.axis_index("s") * NC + lax.axis_index("c")
    seq0 = wid * POS_PER_W

    toks = (tokA, tokB, tokC)
    gts = (gtA, gtB, gtC)
    outs = (o0, o1)
    gos = (go0, go1)
    poss = (pos0, pos1)
    gps = (gp0, gp1)

    pltpu.sync_copy(ids_hbm.at[wid], idx_v)
    pltpu.sync_copy(mod_hbm, mod_v)

    def zero_body(j, _):
        acc_v[j // G, pl.ds((j % G) * 16, 16)] = jnp.zeros((16,), jnp.float32)
        return 0
    lax.fori_loop(0, B * G, zero_body, 0)

    def gather_cp(c, k):
        return pltpu.make_async_copy(
            tok_hbm.at[idx_v.at[pl.ds(c * B * CH, B * CH)]], toks[k], gts[k])

    def pos_cp(c, j):
        return pltpu.make_async_copy(
            pos_hbm.at[pl.ds(seq0 + c * CH, CH)], poss[j], gps[j])

    def out_cps(c, j):
        buf, sem = outs[j], gos[j]
        return [
            pltpu.make_async_copy(
                buf.at[pl.ds(bi * CH, CH)],
                out_hbm.at[bi, pl.ds(seq0 + c * CH, CH)], sem)
            for bi in range(B)
        ]

    def compute_chunk(tok_v, pos_v, out_v):
        def g_body(g, _):
            o = g * 16
            m = mod_v[pl.ds(o, 16)]
            parts = [[None, None] for _ in range(B)]
            for r in range(CH):
                p = pos_v[r, pl.ds(o, 16)]
                pm = p + m
                for bi in range(B):
                    t = tok_v[bi * CH + r, pl.ds(o, 16)]
                    e = t * SCALE + pm
                    out_v[bi * CH + r, pl.ds(o, 16)] = e
                    k = r % 2
                    pb = parts[bi]
                    pb[k] = e if pb[k] is None else pb[k] + e
            for bi in range(B):
                acc_v[bi, pl.ds(o, 16)] = (
                    acc_v[bi, pl.ds(o, 16)] + (parts[bi][0] + parts[bi][1])
                )
            return 0
        lax.fori_loop(0, G, g_body, 0)

    gather_cp(0, 0).start()
    pos_cp(0, 0).start()
    gather_cp(1, 1).start()
    pos_cp(1, 1).start()

    for c in range(NCHUNK):
        k = c % 3
        j = c % 2
        gather_cp(c, k).wait()
        pos_cp(c, j).wait()
        if c >= 2:
            for cp in out_cps(c - 2, j):
                cp.wait()
        compute_chunk(toks[k], poss[j], outs[j])
        for cp in out_cps(c, j):
            cp.start()
        if c + 2 < NCHUNK:
            gather_cp(c + 2, (c + 2) % 3).start()
            pos_cp(c + 2, j).start()

    for c in range(NCHUNK - 2, NCHUNK):
        for cp in out_cps(c, c % 2):
            cp.wait()

    for bi in range(B):
        pltpu.sync_copy(acc_v.at[bi], part_hbm.at[bi, wid])


def _ctx_body(part_ref, w_ref, b_ref, out_ref):
    mean = jnp.sum(part_ref[...], axis=1) * (1.0 / S)
    out_ref[...] = (
        jnp.dot(mean, w_ref[...], preferred_element_type=jnp.float32)
        + b_ref[...]
    )


_ctx_proj = pl.pallas_call(
    _ctx_body,
    out_shape=jax.ShapeDtypeStruct((B, D), jnp.float32),
)


def kernel(input_ids, modality, token_embed, pos_embed, modality_embed, W_ctx, b_ctx):
    ids = (
        input_ids.astype(jnp.int32)
        .reshape(B, NW, NCHUNK, CH)
        .transpose(1, 2, 0, 3)
        .reshape(NW, NCHUNK * B * CH)
    )
    mod_row = lax.dynamic_index_in_dim(
        modality_embed, modality, axis=0, keepdims=False
    )
    emb, part = _sc_embed(ids, token_embed, pos_embed, mod_row)
    context = _ctx_proj(part, W_ctx, b_ctx.reshape(1, D))
    return emb, context


# speedup vs baseline: 1.7092x; 1.7092x over previous
import functools
import math

import jax
import jax.numpy as jnp
from jax import lax
from jax.experimental import pallas as pl
from jax.experimental.pallas import tpu as pltpu
from jax.experimental.pallas import tpu_sc as plsc

B = 4
S = 4096
D = 1024
SCALE = math.sqrt(D)

NC = 2
NS = 16
NW = NC * NS
POS_PER_W = S // NW
CH = 8
NCHUNK = POS_PER_W // CH
G = D // 16

_mesh = plsc.VectorSubcoreMesh(
    core_axis_name="c", subcore_axis_name="s", num_cores=NC, num_subcores=NS
)


@functools.partial(
    pl.kernel,
    out_type=(
        jax.ShapeDtypeStruct((B, S, D), jnp.float32),
        jax.ShapeDtypeStruct((B, NW, D), jnp.float32),
    ),
    mesh=_mesh,
    scratch_types=[
        pltpu.VMEM((NCHUNK * B * CH,), jnp.int32),
        pltpu.VMEM((B * CH, D), jnp.float32),
        pltpu.VMEM((B * CH, D), jnp.float32),
        pltpu.VMEM((B * CH, D), jnp.float32),
        pltpu.VMEM((CH, D), jnp.float32),
        pltpu.VMEM((CH, D), jnp.float32),
        pltpu.VMEM((1, D), jnp.float32),
        pltpu.VMEM((B, D), jnp.float32),
        pltpu.VMEM((1,), jnp.int32),
        pltpu.SemaphoreType.DMA,
        pltpu.SemaphoreType.DMA,
        pltpu.SemaphoreType.DMA,
        pltpu.SemaphoreType.DMA,
        pltpu.SemaphoreType.DMA,
        pltpu.SemaphoreType.DMA,
        pltpu.SemaphoreType.DMA,
        pltpu.SemaphoreType.DMA,
    ],
)
def _sc_embed(ids_hbm, tok_hbm, pos_hbm, modality_hbm, modtab_hbm,
              out_hbm, part_hbm,
              idx_v, tokA, tokB, tokC, pos0, pos1, mod_v, acc_v, mid_s,
              gtA, gtB, gtC, gp0, gp1, goA, goB, goC):
    wid = lax.axis_index("s") * NC + lax.axis_index("c")
    seq0 = wid * POS_PER_W

    toks = (tokA, tokB, tokC)
    gts = (gtA, gtB, gtC)
    gos = (goA, goB, goC)
    poss = (pos0, pos1)
    gps = (gp0, gp1)

    pltpu.sync_copy(ids_hbm.at[wid], idx_v)
    pltpu.sync_copy(modality_hbm, mid_s)
    pltpu.sync_copy(modtab_hbm.at[mid_s], mod_v)

    def zero_body(j, _):
        acc_v[j // G, pl.ds((j % G) * 16, 16)] = jnp.zeros((16,), jnp.float32)
        return 0
    lax.fori_loop(0, B * G, zero_body, 0)

    def gather_cp(c, k):
        return pltpu.make_async_copy(
            tok_hbm.at[idx_v.at[pl.ds(c * B * CH, B * CH)]], toks[k], gts[k])

    def pos_cp(c, j):
        return pltpu.make_async_copy(
            pos_hbm.at[pl.ds(seq0 + c * CH, CH)], poss[j], gps[j])

    def out_cps(c, k):
        buf, sem = toks[k], gos[k]
        return [
            pltpu.make_async_copy(
                buf.at[pl.ds(bi * CH, CH)],
                out_hbm.at[bi, pl.ds(seq0 + c * CH, CH)], sem)
            for bi in range(B)
        ]

    def compute_chunk(tok_v, pos_v):
        def g_body(g, _):
            o = g * 16
            m = mod_v[0, pl.ds(o, 16)]
            parts = [[None, None] for _ in range(B)]
            for r in range(CH):
                p = pos_v[r, pl.ds(o, 16)]
                pm = p + m
                for bi in range(B):
                    t = tok_v[bi * CH + r, pl.ds(o, 16)]
                    e = t * SCALE + pm
                    tok_v[bi * CH + r, pl.ds(o, 16)] = e
                    k = r % 2
                    pb = parts[bi]
                    pb[k] = e if pb[k] is None else pb[k] + e
            for bi in range(B):
                acc_v[bi, pl.ds(o, 16)] = (
                    acc_v[bi, pl.ds(o, 16)] + (parts[bi][0] + parts[bi][1])
                )
            return 0
        lax.fori_loop(0, G, g_body, 0)

    gather_cp(0, 0).start()
    pos_cp(0, 0).start()
    gather_cp(1, 1).start()
    pos_cp(1, 1).start()

    for c in range(NCHUNK):
        k = c % 3
        j = c % 2
        gather_cp(c, k).wait()
        pos_cp(c, j).wait()
        compute_chunk(toks[k], poss[j])
        for cp in out_cps(c, k):
            cp.start()
        if c + 2 < NCHUNK:
            k2 = (c + 2) % 3
            if c - 1 >= 0:
                for cp in out_cps(c - 1, k2):
                    cp.wait()
            gather_cp(c + 2, k2).start()
            pos_cp(c + 2, j).start()

    for c in range(NCHUNK - 3, NCHUNK):
        for cp in out_cps(c, c % 3):
            cp.wait()

    for bi in range(B):
        pltpu.sync_copy(acc_v.at[bi], part_hbm.at[bi, wid])


def _ctx_body(part_ref, w_ref, b_ref, out_ref):
    mean = jnp.sum(part_ref[...], axis=1) * (1.0 / S)
    out_ref[...] = (
        jnp.dot(mean, w_ref[...], preferred_element_type=jnp.float32)
        + b_ref[...]
    )


_ctx_proj = pl.pallas_call(
    _ctx_body,
    out_shape=jax.ShapeDtypeStruct((B, D), jnp.float32),
)


def kernel(input_ids, modality, token_embed, pos_embed, modality_embed, W_ctx, b_ctx):
    ids = (
        input_ids.astype(jnp.int32)
        .reshape(B, NW, NCHUNK, CH)
        .transpose(1, 2, 0, 3)
        .reshape(NW, NCHUNK * B * CH)
    )
    mid = jnp.asarray(modality, jnp.int32).reshape(1)
    emb, part = _sc_embed(ids, token_embed, pos_embed, mid, modality_embed)
    context = _ctx_proj(part, W_ctx, b_ctx.reshape(1, D))
    return emb, context
